# Initial kernel scaffold; baseline (speedup 1.0000x reference)
import jax
import jax.numpy as jnp
from jax.experimental import pallas as pl


def kernel(x, edge_index, node_time, W_in, b_in, agent_emb, W_an, b_an, W_q, b_q, W_k, b_k, w_a, b_a, W_am1, b_am1, W_am2, b_am2, g_al, b_al, W_nm1, b_nm1, W_nm2, b_nm2, g_nl, b_nl, W_mv, b_mv, W_cm1, b_cm1, W_cm2, b_cm2, g_cl, b_cl):
    raise NotImplementedError("write your pallas kernel here")


# speedup vs baseline: 6.1783x; 6.1783x over previous
import functools

import jax
import jax.numpy as jnp
from jax import lax
from jax.experimental import pallas as pl
from jax.experimental.pallas import tpu as pltpu
from jax.experimental.pallas import tpu_sc as plsc

NN = 10000
FF = 128
DD = 64
KK = 32
PP = 4
EE = 160000

NC, NS = 2, 16
NW = NC * NS
NP = 10240
CA = NP // NW
RT = NP // NS
EP = 163840
ECW = EP // NW
ESUB = 512
BB = 640
NB = NP // BB

_I = False


def _lrelu(v, s):
    return jnp.where(v >= 0, v, s * v)


def _ln(v, g, b):
    m = jnp.mean(v, axis=-1, keepdims=True)
    var = jnp.mean((v - m) ** 2, axis=-1, keepdims=True)
    return (v - m) / jnp.sqrt(var + 1e-5) * g + b


def _full(shape):
    nd = len(shape)
    return pl.BlockSpec(shape, lambda i: (0,) * nd)


def _rows(cols):
    return pl.BlockSpec((BB, cols), lambda i: (i, 0))


def _init_body(x, W_in, b_in, W_an, b_an, Wk1, emb, W_q, b_q, Wk2, b_k,
               h_o, s_o, q_o, t_o, ag_o):
    h = jnp.dot(x[...], W_in[...], preferred_element_type=jnp.float32) + b_in[...]
    hp = _lrelu(jnp.dot(h, W_an[...], preferred_element_type=jnp.float32) + b_an[...], 0.2)
    s_o[...] = jnp.dot(hp, Wk1[...], preferred_element_type=jnp.float32)
    h_o[...] = h
    ag = jnp.broadcast_to(emb[...], (BB, DD))
    q_o[...] = jnp.dot(ag, W_q[...], preferred_element_type=jnp.float32) + b_q[...]
    t_o[...] = jnp.dot(ag, Wk2[...], preferred_element_type=jnp.float32) + b_k[...]
    ag_o[...] = ag


def _tc_init(xp, W_in, b_in, W_an, b_an, Wk1, emb, W_q, b_q, Wk2, b_k):
    f32 = jnp.float32
    outs = [jax.ShapeDtypeStruct((NP, DD), f32)] * 5
    return pl.pallas_call(
        _init_body,
        grid=(NB,),
        in_specs=[_rows(FF), _full((FF, DD)), _full((1, DD)), _full((DD, DD)),
                  _full((1, DD)), _full((DD, DD)), _full((1, DD)),
                  _full((DD, DD)), _full((1, DD)), _full((DD, DD)), _full((1, DD))],
        out_specs=[_rows(DD)] * 5,
        out_shape=outs,
        interpret=_I,
    )(xp, W_in, b_in, W_an, b_an, Wk1, emb, W_q, b_q, Wk2, b_k)


def _sel_body(G, neighT, cnt, cur, noise, q, t, A, Bc, nxt_o, logp_o):
    qv = q[...]
    c = jnp.sum(qv * t[...], axis=-1, keepdims=True)
    cols = []
    for k in range(KK):
        cols.append(jnp.sum(G[k] * qv, axis=-1, keepdims=True))
    raw = jnp.concatenate(cols, axis=1)
    lg = (raw + c) * A[0, 0] + Bc[0, 0]
    kio = lax.broadcasted_iota(jnp.int32, (BB, KK), 1)
    cntv = cnt[...]
    lg = jnp.where(kio < cntv, lg, -1e9)
    y = lg + noise[...]
    mx = jnp.max(y, axis=-1, keepdims=True)
    ch = jnp.min(jnp.where(y == mx, kio, KK), axis=-1, keepdims=True)
    m2 = jnp.max(lg, axis=-1, keepdims=True)
    lse = m2 + jnp.log(jnp.sum(jnp.exp(lg - m2), axis=-1, keepdims=True))
    sel = kio == ch
    lgch = jnp.sum(jnp.where(sel, lg, 0.0), axis=-1, keepdims=True)
    nxtv = jnp.sum(jnp.where(sel, neighT[...], 0), axis=-1, keepdims=True)
    has = cntv > 0
    nxt_o[...] = jnp.where(has, nxtv, cur[...])
    logp_o[...] = jnp.where(has, lgch - lse, 0.0)


def _tc_select(G, neighT, cnt2, cur2, noise, q, t, A, Bc):
    return pl.pallas_call(
        _sel_body,
        grid=(NB,),
        in_specs=[pl.BlockSpec((KK, BB, DD), lambda i: (0, i, 0)),
                  _rows(KK), _rows(1), _rows(1), _rows(KK), _rows(DD),
                  _rows(DD), _full((1, 1)), _full((1, 1))],
        out_specs=[_rows(1), _rows(1)],
        out_shape=[jax.ShapeDtypeStruct((NP, 1), jnp.int32),
                   jax.ShapeDtypeStruct((NP, 1), jnp.float32)],
        interpret=_I,
    )(G, neighT, cnt2, cur2, noise, q, t, A, Bc)


def _agent_body(ag, hsel, W_am1, b_am1, W_am2, b_am2, g_al, b_al,
                W_nm1, b_nm1, W_nm2, b_nm2, ag_o, upd_o):
    agv = ag[...]
    hs = hsel[...]
    a_in = jnp.concatenate([agv, hs], axis=-1)
    z = jnp.dot(_lrelu(jnp.dot(a_in, W_am1[...], preferred_element_type=jnp.float32)
                       + b_am1[...], 0.01),
                W_am2[...], preferred_element_type=jnp.float32) + b_am2[...]
    ag2 = _ln(agv + z, g_al[...], b_al[...])
    n_in = jnp.concatenate([hs, ag2], axis=-1)
    upd_o[...] = jnp.dot(_lrelu(jnp.dot(n_in, W_nm1[...], preferred_element_type=jnp.float32)
                                + b_nm1[...], 0.01),
                         W_nm2[...], preferred_element_type=jnp.float32) + b_nm2[...]
    ag_o[...] = ag2


def _tc_agent(ag, hsel, W_am1, b_am1, W_am2, b_am2, g_al, b_al,
              W_nm1, b_nm1, W_nm2, b_nm2):
    f32 = jnp.float32
    return pl.pallas_call(
        _agent_body,
        grid=(NB,),
        in_specs=[_rows(DD), _rows(DD), _full((2 * DD, 2 * DD)), _full((1, 2 * DD)),
                  _full((2 * DD, DD)), _full((1, DD)), _full((1, DD)), _full((1, DD)),
                  _full((2 * DD, 2 * DD)), _full((1, 2 * DD)), _full((2 * DD, DD)),
                  _full((1, DD))],
        out_specs=[_rows(DD)] * 2,
        out_shape=[jax.ShapeDtypeStruct((NP, DD), f32)] * 2,
        interpret=_I,
    )(ag, hsel, W_am1, b_am1, W_am2, b_am2, g_al, b_al, W_nm1, b_nm1, W_nm2, b_nm2)


def _mid_body(h, delta, g_nl, b_nl, W_mv, b_mv, hm_o, msg_o):
    hm = _ln(h[...] + delta[0] + delta[1], g_nl[...], b_nl[...])
    hm_o[...] = hm
    msg_o[...] = _lrelu(jnp.dot(hm, W_mv[...], preferred_element_type=jnp.float32)
                        + b_mv[...], 0.2)


def _tc_mid(h, delta, g_nl, b_nl, W_mv, b_mv):
    f32 = jnp.float32
    return pl.pallas_call(
        _mid_body,
        grid=(NB,),
        in_specs=[_rows(DD), pl.BlockSpec((NC, BB, DD), lambda i: (0, i, 0)),
                  _full((1, DD)), _full((1, DD)), _full((DD, DD)), _full((1, DD))],
        out_specs=[_rows(DD)] * 2,
        out_shape=[jax.ShapeDtypeStruct((NP, DD), f32)] * 2,
        interpret=_I,
    )(h, delta, g_nl, b_nl, W_mv, b_mv)


def _final_body(hm, agg, ag, W_cm1, b_cm1, W_cm2, b_cm2, g_cl, b_cl,
                W_an, b_an, Wk1, W_q, b_q, Wk2, b_k,
                h_o, s_o, q_o, t_o):
    hmv = hm[...]
    c_in = jnp.concatenate([hmv, agg[0] + agg[1]], axis=-1)
    z = jnp.dot(_lrelu(jnp.dot(c_in, W_cm1[...], preferred_element_type=jnp.float32)
                       + b_cm1[...], 0.01),
                W_cm2[...], preferred_element_type=jnp.float32) + b_cm2[...]
    h = _ln(hmv + z, g_cl[...], b_cl[...])
    h_o[...] = h
    hp = _lrelu(jnp.dot(h, W_an[...], preferred_element_type=jnp.float32) + b_an[...], 0.2)
    s_o[...] = jnp.dot(hp, Wk1[...], preferred_element_type=jnp.float32)
    agv = ag[...]
    q_o[...] = jnp.dot(agv, W_q[...], preferred_element_type=jnp.float32) + b_q[...]
    t_o[...] = jnp.dot(agv, Wk2[...], preferred_element_type=jnp.float32) + b_k[...]


def _tc_final(hm, agg, ag, W_cm1, b_cm1, W_cm2, b_cm2, g_cl, b_cl,
              W_an, b_an, Wk1, W_q, b_q, Wk2, b_k):
    f32 = jnp.float32
    return pl.pallas_call(
        _final_body,
        grid=(NB,),
        in_specs=[_rows(DD), pl.BlockSpec((NC, BB, DD), lambda i: (0, i, 0)),
                  _rows(DD), _full((2 * DD, 2 * DD)), _full((1, 2 * DD)),
                  _full((2 * DD, DD)), _full((1, DD)), _full((1, DD)), _full((1, DD)),
                  _full((DD, DD)), _full((1, DD)), _full((DD, DD)),
                  _full((DD, DD)), _full((1, DD)), _full((DD, DD)), _full((1, DD))],
        out_specs=[_rows(DD)] * 4,
        out_shape=[jax.ShapeDtypeStruct((NP, DD), f32)] * 4,
        interpret=_I,
    )(hm, agg, ag, W_cm1, b_cm1, W_cm2, b_cm2, g_cl, b_cl,
      W_an, b_an, Wk1, W_q, b_q, Wk2, b_k)


@functools.cache
def _mesh():
    return plsc.VectorSubcoreMesh(core_axis_name="c", subcore_axis_name="s",
                                  num_cores=NC, num_subcores=NS)


def _wid():
    return lax.axis_index("s") * NC + lax.axis_index("c")


def _sc_gather_body(cur_h, lo_h, hi_h, dst_h, s_h,
                    cnt_h, neigh_h, G_h,
                    curv, basev, hiv, cntv, idxv, neighv, rows, sem):
    a0 = _wid() * CA
    pltpu.sync_copy(cur_h.at[pl.ds(a0, CA)], curv)
    pltpu.async_copy(lo_h.at[curv], basev, sem).wait()
    pltpu.async_copy(hi_h.at[curv], hiv, sem).wait()

    def cnt_chunk(j, _):
        sl = pl.ds(j * 16, 16)
        cntv[sl] = hiv[sl] - basev[sl]
        return 0
    lax.fori_loop(0, CA // 16, cnt_chunk, 0)
    pltpu.sync_copy(cntv, cnt_h.at[pl.ds(a0, CA)])

    def per_k(k, _):
        def idx_chunk(j, _):
            sl = pl.ds(j * 16, 16)
            idxv[sl] = jnp.clip(basev[sl] + k, 0, EE - 1)
            return 0
        lax.fori_loop(0, CA // 16, idx_chunk, 0)
        pltpu.async_copy(dst_h.at[idxv], neighv, sem).wait()
        pltpu.sync_copy(neighv, neigh_h.at[pl.ds(k * NP + a0, CA)])
        pltpu.async_copy(s_h.at[neighv], rows, sem).wait()
        pltpu.sync_copy(rows, G_h.at[k, pl.ds(a0, CA)])
        return 0
    lax.fori_loop(0, KK, per_k, 0)


def _sc_gather(cur, lo_p, hi_p, dst_s, s):
    i32, f32 = jnp.int32, jnp.float32
    f = pl.kernel(
        _sc_gather_body,
        out_type=[jax.ShapeDtypeStruct((NP,), i32),
                  jax.ShapeDtypeStruct((KK * NP,), i32),
                  jax.ShapeDtypeStruct((KK, NP, DD), f32)],
        mesh=_mesh(),
        compiler_params=pltpu.CompilerParams(use_tc_tiling_on_sc=False),
        scratch_types=[pltpu.VMEM((CA,), i32)] * 6
        + [pltpu.VMEM((CA, DD), f32), pltpu.SemaphoreType.DMA],
        interpret=_I,
    )
    return f(cur, lo_p, hi_p, dst_s, s)


def _sc_rowgather_body(idx_h, tab_h, out_h, idxv, rows, sem):
    a0 = _wid() * CA
    pltpu.sync_copy(idx_h.at[pl.ds(a0, CA)], idxv)
    pltpu.async_copy(tab_h.at[idxv], rows, sem).wait()
    pltpu.sync_copy(rows, out_h.at[pl.ds(a0, CA)])


def _sc_rowgather(idx, tab):
    f = pl.kernel(
        _sc_rowgather_body,
        out_type=[jax.ShapeDtypeStruct((NP, DD), jnp.float32)],
        mesh=_mesh(),
        compiler_params=pltpu.CompilerParams(use_tc_tiling_on_sc=False),
        scratch_types=[pltpu.VMEM((CA,), jnp.int32),
                       pltpu.VMEM((CA, DD), jnp.float32),
                       pltpu.SemaphoreType.DMA],
        interpret=_I,
    )
    return f(idx, tab)[0]


def _sc_scatter_body(idx_h, val_h, zero_h, out_h, idxv, rows, acc, sem):
    cid = lax.axis_index("c")
    sid = lax.axis_index("s")
    a0 = _wid() * CA
    r0 = sid * RT
    pltpu.sync_copy(zero_h.at[pl.ds(r0, RT)], acc.at[pl.ds(r0, RT)])
    plsc.subcore_barrier()
    pltpu.sync_copy(idx_h.at[pl.ds(a0, CA)], idxv)
    pltpu.sync_copy(val_h.at[pl.ds(a0, CA)], rows)
    pltpu.async_copy(rows, acc.at[idxv], sem, add=True).wait()
    plsc.subcore_barrier()
    pltpu.sync_copy(acc.at[pl.ds(r0, RT)], out_h.at[cid, pl.ds(r0, RT)])


def _sc_scatter_add(idx, val, zeros):
    f = pl.kernel(
        _sc_scatter_body,
        out_type=[jax.ShapeDtypeStruct((NC, NP, DD), jnp.float32)],
        mesh=_mesh(),
        compiler_params=pltpu.CompilerParams(use_tc_tiling_on_sc=False),
        scratch_types=[pltpu.VMEM((CA,), jnp.int32),
                       pltpu.VMEM((CA, DD), jnp.float32),
                       pltpu.VMEM_SHARED((NP, DD), jnp.float32),
                       pltpu.SemaphoreType.DMA],
        interpret=_I,
    )
    return f(idx, val, zeros)[0]


def _sc_edge_body(src_h, dst_h, msg_h, zero_h, out_h,
                  sidxv, didxv, rows, acc, sem):
    cid = lax.axis_index("c")
    sid = lax.axis_index("s")
    e0 = _wid() * ECW
    r0 = sid * RT
    pltpu.sync_copy(zero_h.at[pl.ds(r0, RT)], acc.at[pl.ds(r0, RT)])
    plsc.subcore_barrier()

    def sub(it, _):
        eo = e0 + it * ESUB
        pltpu.sync_copy(src_h.at[pl.ds(eo, ESUB)], sidxv)
        pltpu.async_copy(msg_h.at[sidxv], rows, sem).wait()
        pltpu.sync_copy(dst_h.at[pl.ds(eo, ESUB)], didxv)
        pltpu.async_copy(rows, acc.at[didxv], sem, add=True).wait()
        return 0
    lax.fori_loop(0, ECW // ESUB, sub, 0)
    plsc.subcore_barrier()
    pltpu.sync_copy(acc.at[pl.ds(r0, RT)], out_h.at[cid, pl.ds(r0, RT)])


def _sc_edge_agg(src_e, dst_e, msg, zeros):
    f = pl.kernel(
        _sc_edge_body,
        out_type=[jax.ShapeDtypeStruct((NC, NP, DD), jnp.float32)],
        mesh=_mesh(),
        compiler_params=pltpu.CompilerParams(use_tc_tiling_on_sc=False),
        scratch_types=[pltpu.VMEM((ESUB,), jnp.int32),
                       pltpu.VMEM((ESUB,), jnp.int32),
                       pltpu.VMEM((ESUB, DD), jnp.float32),
                       pltpu.VMEM_SHARED((NP, DD), jnp.float32),
                       pltpu.SemaphoreType.DMA],
        interpret=_I,
    )
    return f(src_e, dst_e, msg, zeros)[0]


def kernel(x, edge_index, node_time, W_in, b_in, agent_emb, W_an, b_an,
           W_q, b_q, W_k, b_k, w_a, b_a, W_am1, b_am1, W_am2, b_am2,
           g_al, b_al, W_nm1, b_nm1, W_nm2, b_nm2, g_nl, b_nl, W_mv, b_mv,
           W_cm1, b_cm1, W_cm2, b_cm2, g_cl, b_cl):
    del node_time
    i32, f32 = jnp.int32, jnp.float32

    src = edge_index[0]
    dst = edge_index[1]
    order = jnp.argsort(src)
    src_s = src[order].astype(i32)
    dst_s = dst[order].astype(i32)
    nodes = jnp.arange(NN, dtype=i32)
    lo = jnp.searchsorted(src_s, nodes, side='left').astype(i32)
    hi = jnp.searchsorted(src_s, nodes, side='right').astype(i32)
    lo_p = jnp.concatenate([lo, jnp.zeros((NP - NN,), i32)])
    hi_p = jnp.concatenate([hi, jnp.zeros((NP - NN,), i32)])
    src_e = jnp.concatenate([src_s, jnp.zeros((EP - EE,), i32)])
    dst_e = jnp.concatenate([dst_s, jnp.full((EP - EE,), NP - 1, i32)])

    xp = jnp.concatenate([x, jnp.zeros((NP - NN, FF), f32)])
    zeros = jnp.zeros((NP, DD), f32)

    noises = []
    for step in range(PP):
        gkey = jax.random.fold_in(jax.random.key(42), step)
        gu = jax.random.uniform(gkey, (NN, KK), minval=1e-6, maxval=1.0 - 1e-6)
        gn = -jnp.log(-jnp.log(gu))
        noises.append(jnp.concatenate([gn, jnp.zeros((NP - NN, KK), f32)]))

    scale = 1.0 / float(DD) ** 0.5
    A = (w_a[0] * scale).reshape(1, 1).astype(f32)
    Bc = b_a[0].reshape(1, 1).astype(f32)

    r1 = lambda v: v.reshape(1, -1)
    Wk1 = W_k[:DD]
    Wk2 = W_k[DD:]

    h, s, q, t, agent = _tc_init(xp, W_in, r1(b_in), W_an, r1(b_an), Wk1,
                                 r1(agent_emb), W_q, r1(b_q), Wk2, r1(b_k))

    cur = jnp.concatenate([nodes, jnp.full((NP - NN,), NP - 1, i32)])
    visited = [nodes]
    logps = []
    for step in range(PP):
        cnt, neigh, G = _sc_gather(cur, lo_p, hi_p, dst_s, s)
        neigh = neigh.reshape(KK, NP)
        nxt2, logp2 = _tc_select(G, neigh.T, cnt[:, None], cur[:, None],
                                 noises[step], q, t, A, Bc)
        nxt = nxt2[:, 0]
        visited.append(nxt[:NN])
        logps.append(logp2[:NN, 0])
        if step == PP - 1:
            break
        hsel = _sc_rowgather(nxt, h)
        agent, upd = _tc_agent(agent, hsel, W_am1, r1(b_am1), W_am2, r1(b_am2),
                               r1(g_al), r1(b_al), W_nm1, r1(b_nm1), W_nm2,
                               r1(b_nm2))
        delta = _sc_scatter_add(nxt, upd, zeros)
        hm, msg = _tc_mid(h, delta, r1(g_nl), r1(b_nl), W_mv, r1(b_mv))
        agg = _sc_edge_agg(src_e, dst_e, msg, zeros)
        h, s, q, t = _tc_final(hm, agg, agent, W_cm1, r1(b_cm1), W_cm2,
                               r1(b_cm2), r1(g_cl), r1(b_cl), W_an, r1(b_an),
                               Wk1, W_q, r1(b_q), Wk2, r1(b_k))
        cur = nxt

    return (jnp.stack(visited, axis=1), jnp.stack(logps, axis=1))


# speedup vs baseline: 7.1420x; 1.1560x over previous
import functools

import jax
import jax.numpy as jnp
from jax import lax
from jax.experimental import pallas as pl
from jax.experimental.pallas import tpu as pltpu
from jax.experimental.pallas import tpu_sc as plsc

NN = 10000
FF = 128
DD = 64
KK = 32
PP = 4
EE = 160000

NC, NS = 2, 16
NW = NC * NS
NP = 10240
CA = NP // NW
RT = NP // NS
EP = 163840
ECW = EP // NW
ESUB = 512
BB = 640
NB = NP // BB

_I = False


def _lrelu(v, s):
    return jnp.where(v >= 0, v, s * v)


def _ln(v, g, b):
    m = jnp.mean(v, axis=-1, keepdims=True)
    var = jnp.mean((v - m) ** 2, axis=-1, keepdims=True)
    return (v - m) / jnp.sqrt(var + 1e-5) * g + b


def _full(shape):
    nd = len(shape)
    return pl.BlockSpec(shape, lambda i: (0,) * nd)


def _rows(cols):
    return pl.BlockSpec((BB, cols), lambda i: (i, 0))


def _init_body(x, W_in, b_in, W_an, b_an, Wk1, emb, W_q, b_q, Wk2, b_k,
               h_o, s_o, q_o, t_o, ag_o):
    h = jnp.dot(x[...], W_in[...], preferred_element_type=jnp.float32) + b_in[...]
    hp = _lrelu(jnp.dot(h, W_an[...], preferred_element_type=jnp.float32) + b_an[...], 0.2)
    s_o[...] = jnp.dot(hp, Wk1[...], preferred_element_type=jnp.float32)
    h_o[...] = h
    ag = jnp.broadcast_to(emb[...], (BB, DD))
    q_o[...] = jnp.dot(ag, W_q[...], preferred_element_type=jnp.float32) + b_q[...]
    t_o[...] = jnp.dot(ag, Wk2[...], preferred_element_type=jnp.float32) + b_k[...]
    ag_o[...] = ag


def _tc_init(xp, W_in, b_in, W_an, b_an, Wk1, emb, W_q, b_q, Wk2, b_k):
    f32 = jnp.float32
    outs = [jax.ShapeDtypeStruct((NP, DD), f32)] * 5
    return pl.pallas_call(
        _init_body,
        grid=(NB,),
        in_specs=[_rows(FF), _full((FF, DD)), _full((1, DD)), _full((DD, DD)),
                  _full((1, DD)), _full((DD, DD)), _full((1, DD)),
                  _full((DD, DD)), _full((1, DD)), _full((DD, DD)), _full((1, DD))],
        out_specs=[_rows(DD)] * 5,
        out_shape=outs,
        interpret=_I,
    )(xp, W_in, b_in, W_an, b_an, Wk1, emb, W_q, b_q, Wk2, b_k)


def _sel_body(G, neighT, cnt, cur, noise, q, t, A, Bc, nxt_o, logp_o):
    qv = q[...]
    c = jnp.sum(qv * t[...], axis=-1, keepdims=True)
    cols = []
    for k in range(KK):
        cols.append(jnp.sum(G[k] * qv, axis=-1, keepdims=True))
    raw = jnp.concatenate(cols, axis=1)
    lg = (raw + c) * A[0, 0] + Bc[0, 0]
    kio = lax.broadcasted_iota(jnp.int32, (BB, KK), 1)
    cntv = cnt[...]
    lg = jnp.where(kio < cntv, lg, -1e9)
    y = lg + noise[...]
    mx = jnp.max(y, axis=-1, keepdims=True)
    ch = jnp.min(jnp.where(y == mx, kio, KK), axis=-1, keepdims=True)
    m2 = jnp.max(lg, axis=-1, keepdims=True)
    lse = m2 + jnp.log(jnp.sum(jnp.exp(lg - m2), axis=-1, keepdims=True))
    sel = kio == ch
    lgch = jnp.sum(jnp.where(sel, lg, 0.0), axis=-1, keepdims=True)
    nxtv = jnp.sum(jnp.where(sel, neighT[...], 0), axis=-1, keepdims=True)
    has = cntv > 0
    nxt_o[...] = jnp.where(has, nxtv, cur[...])
    logp_o[...] = jnp.where(has, lgch - lse, 0.0)


def _tc_select(G, neighT, cnt2, cur2, noise, q, t, A, Bc):
    return pl.pallas_call(
        _sel_body,
        grid=(NB,),
        in_specs=[pl.BlockSpec((KK, BB, DD), lambda i: (0, i, 0)),
                  _rows(KK), _rows(1), _rows(1), _rows(KK), _rows(DD),
                  _rows(DD), _full((1, 1)), _full((1, 1))],
        out_specs=[_rows(1), _rows(1)],
        out_shape=[jax.ShapeDtypeStruct((NP, 1), jnp.int32),
                   jax.ShapeDtypeStruct((NP, 1), jnp.float32)],
        interpret=_I,
    )(G, neighT, cnt2, cur2, noise, q, t, A, Bc)


def _agent_body(ag, hsel, W_am1, b_am1, W_am2, b_am2, g_al, b_al,
                W_nm1, b_nm1, W_nm2, b_nm2, ag_o, upd_o):
    agv = ag[...]
    hs = hsel[...]
    a_in = jnp.concatenate([agv, hs], axis=-1)
    z = jnp.dot(_lrelu(jnp.dot(a_in, W_am1[...], preferred_element_type=jnp.float32)
                       + b_am1[...], 0.01),
                W_am2[...], preferred_element_type=jnp.float32) + b_am2[...]
    ag2 = _ln(agv + z, g_al[...], b_al[...])
    n_in = jnp.concatenate([hs, ag2], axis=-1)
    upd_o[...] = jnp.dot(_lrelu(jnp.dot(n_in, W_nm1[...], preferred_element_type=jnp.float32)
                                + b_nm1[...], 0.01),
                         W_nm2[...], preferred_element_type=jnp.float32) + b_nm2[...]
    ag_o[...] = ag2


def _tc_agent(ag, hsel, W_am1, b_am1, W_am2, b_am2, g_al, b_al,
              W_nm1, b_nm1, W_nm2, b_nm2):
    f32 = jnp.float32
    return pl.pallas_call(
        _agent_body,
        grid=(NB,),
        in_specs=[_rows(DD), _rows(DD), _full((2 * DD, 2 * DD)), _full((1, 2 * DD)),
                  _full((2 * DD, DD)), _full((1, DD)), _full((1, DD)), _full((1, DD)),
                  _full((2 * DD, 2 * DD)), _full((1, 2 * DD)), _full((2 * DD, DD)),
                  _full((1, DD))],
        out_specs=[_rows(DD)] * 2,
        out_shape=[jax.ShapeDtypeStruct((NP, DD), f32)] * 2,
        interpret=_I,
    )(ag, hsel, W_am1, b_am1, W_am2, b_am2, g_al, b_al, W_nm1, b_nm1, W_nm2, b_nm2)


def _mid_body(h, delta, g_nl, b_nl, W_mv, b_mv, hm_o, msg_o):
    hm = _ln(h[...] + delta[0] + delta[1], g_nl[...], b_nl[...])
    hm_o[...] = hm
    msg_o[...] = _lrelu(jnp.dot(hm, W_mv[...], preferred_element_type=jnp.float32)
                        + b_mv[...], 0.2)


def _tc_mid(h, delta, g_nl, b_nl, W_mv, b_mv):
    f32 = jnp.float32
    return pl.pallas_call(
        _mid_body,
        grid=(NB,),
        in_specs=[_rows(DD), pl.BlockSpec((NC, BB, DD), lambda i: (0, i, 0)),
                  _full((1, DD)), _full((1, DD)), _full((DD, DD)), _full((1, DD))],
        out_specs=[_rows(DD)] * 2,
        out_shape=[jax.ShapeDtypeStruct((NP, DD), f32)] * 2,
        interpret=_I,
    )(h, delta, g_nl, b_nl, W_mv, b_mv)


def _final_body(hm, agg, ag, W_cm1, b_cm1, W_cm2, b_cm2, g_cl, b_cl,
                W_an, b_an, Wk1, W_q, b_q, Wk2, b_k,
                h_o, s_o, q_o, t_o):
    hmv = hm[...]
    c_in = jnp.concatenate([hmv, agg[0] + agg[1]], axis=-1)
    z = jnp.dot(_lrelu(jnp.dot(c_in, W_cm1[...], preferred_element_type=jnp.float32)
                       + b_cm1[...], 0.01),
                W_cm2[...], preferred_element_type=jnp.float32) + b_cm2[...]
    h = _ln(hmv + z, g_cl[...], b_cl[...])
    h_o[...] = h
    hp = _lrelu(jnp.dot(h, W_an[...], preferred_element_type=jnp.float32) + b_an[...], 0.2)
    s_o[...] = jnp.dot(hp, Wk1[...], preferred_element_type=jnp.float32)
    agv = ag[...]
    q_o[...] = jnp.dot(agv, W_q[...], preferred_element_type=jnp.float32) + b_q[...]
    t_o[...] = jnp.dot(agv, Wk2[...], preferred_element_type=jnp.float32) + b_k[...]


def _tc_final(hm, agg, ag, W_cm1, b_cm1, W_cm2, b_cm2, g_cl, b_cl,
              W_an, b_an, Wk1, W_q, b_q, Wk2, b_k):
    f32 = jnp.float32
    return pl.pallas_call(
        _final_body,
        grid=(NB,),
        in_specs=[_rows(DD), pl.BlockSpec((NC, BB, DD), lambda i: (0, i, 0)),
                  _rows(DD), _full((2 * DD, 2 * DD)), _full((1, 2 * DD)),
                  _full((2 * DD, DD)), _full((1, DD)), _full((1, DD)), _full((1, DD)),
                  _full((DD, DD)), _full((1, DD)), _full((DD, DD)),
                  _full((DD, DD)), _full((1, DD)), _full((DD, DD)), _full((1, DD))],
        out_specs=[_rows(DD)] * 4,
        out_shape=[jax.ShapeDtypeStruct((NP, DD), f32)] * 4,
        interpret=_I,
    )(hm, agg, ag, W_cm1, b_cm1, W_cm2, b_cm2, g_cl, b_cl,
      W_an, b_an, Wk1, W_q, b_q, Wk2, b_k)


@functools.cache
def _mesh():
    return plsc.VectorSubcoreMesh(core_axis_name="c", subcore_axis_name="s",
                                  num_cores=NC, num_subcores=NS)


def _wid():
    return lax.axis_index("s") * NC + lax.axis_index("c")


def _sc_gather_body(cur_h, lo_h, hi_h, dst_h, s_h,
                    cnt_h, neigh_h, G_h,
                    curv, basev, hiv, cntv, idxall, neighall,
                    rows0, rows1, semi, semg0, semg1, semw0, semw1):
    w = _wid()
    a0 = w * CA
    pltpu.sync_copy(cur_h.at[pl.ds(a0, CA)], curv)
    pltpu.async_copy(lo_h.at[curv], basev, semi).wait()
    pltpu.async_copy(hi_h.at[curv], hiv, semi).wait()

    def cnt_chunk(j, _):
        sl = pl.ds(j * 16, 16)
        cntv[sl] = hiv[sl] - basev[sl]
        return 0
    lax.fori_loop(0, CA // 16, cnt_chunk, 0)
    pltpu.sync_copy(cntv, cnt_h.at[pl.ds(a0, CA)])

    def idx_k(k, _):
        def idx_chunk(j, _):
            sl = pl.ds(j * 16, 16)
            idxall[pl.ds(k * CA + j * 16, 16)] = jnp.clip(basev[sl] + k, 0, EE - 1)
            return 0
        lax.fori_loop(0, CA // 16, idx_chunk, 0)
        return 0
    lax.fori_loop(0, KK, idx_k, 0)
    pltpu.async_copy(dst_h.at[idxall], neighall, semi).wait()
    pltpu.sync_copy(neighall, neigh_h.at[w])

    bufs = (rows0, rows1)
    gsem = (semg0, semg1)
    wsem = (semw0, semw1)
    gd = [None, None]
    wd = [None, None]
    for k in range(KK):
        b = k % 2
        if wd[b] is not None:
            wd[b].wait()
        gd[b] = pltpu.async_copy(s_h.at[neighall.at[pl.ds(k * CA, CA)]],
                                 bufs[b], gsem[b])
        if k > 0:
            pb = 1 - b
            gd[pb].wait()
            wd[pb] = pltpu.async_copy(bufs[pb], G_h.at[k - 1, pl.ds(a0, CA)],
                                      wsem[pb])
    b = (KK - 1) % 2
    gd[b].wait()
    wd[b] = pltpu.async_copy(bufs[b], G_h.at[KK - 1, pl.ds(a0, CA)], wsem[b])
    wd[1 - b].wait()
    wd[b].wait()


def _sc_gather(cur, lo_p, hi_p, dst_s, s):
    i32, f32 = jnp.int32, jnp.float32
    f = pl.kernel(
        _sc_gather_body,
        out_type=[jax.ShapeDtypeStruct((NP,), i32),
                  jax.ShapeDtypeStruct((NW, KK * CA), i32),
                  jax.ShapeDtypeStruct((KK, NP, DD), f32)],
        mesh=_mesh(),
        compiler_params=pltpu.CompilerParams(use_tc_tiling_on_sc=False),
        scratch_types=[pltpu.VMEM((CA,), i32)] * 4
        + [pltpu.VMEM((KK * CA,), i32)] * 2
        + [pltpu.VMEM((CA, DD), f32)] * 2
        + [pltpu.SemaphoreType.DMA] * 5,
        interpret=_I,
    )
    return f(cur, lo_p, hi_p, dst_s, s)


def _sc_rowgather_body(idx_h, tab_h, out_h, idxv, rows, sem):
    a0 = _wid() * CA
    pltpu.sync_copy(idx_h.at[pl.ds(a0, CA)], idxv)
    pltpu.async_copy(tab_h.at[idxv], rows, sem).wait()
    pltpu.sync_copy(rows, out_h.at[pl.ds(a0, CA)])


def _sc_rowgather(idx, tab):
    f = pl.kernel(
        _sc_rowgather_body,
        out_type=[jax.ShapeDtypeStruct((NP, DD), jnp.float32)],
        mesh=_mesh(),
        compiler_params=pltpu.CompilerParams(use_tc_tiling_on_sc=False),
        scratch_types=[pltpu.VMEM((CA,), jnp.int32),
                       pltpu.VMEM((CA, DD), jnp.float32),
                       pltpu.SemaphoreType.DMA],
        interpret=_I,
    )
    return f(idx, tab)[0]


def _sc_scatter_body(idx_h, val_h, zero_h, out_h, idxv, rows, acc, sem):
    cid = lax.axis_index("c")
    sid = lax.axis_index("s")
    a0 = _wid() * CA
    r0 = sid * RT
    pltpu.sync_copy(zero_h.at[pl.ds(r0, RT)], acc.at[pl.ds(r0, RT)])
    plsc.subcore_barrier()
    pltpu.sync_copy(idx_h.at[pl.ds(a0, CA)], idxv)
    pltpu.sync_copy(val_h.at[pl.ds(a0, CA)], rows)
    pltpu.async_copy(rows, acc.at[idxv], sem, add=True).wait()
    plsc.subcore_barrier()
    pltpu.sync_copy(acc.at[pl.ds(r0, RT)], out_h.at[cid, pl.ds(r0, RT)])


def _sc_scatter_add(idx, val, zeros):
    f = pl.kernel(
        _sc_scatter_body,
        out_type=[jax.ShapeDtypeStruct((NC, NP, DD), jnp.float32)],
        mesh=_mesh(),
        compiler_params=pltpu.CompilerParams(use_tc_tiling_on_sc=False),
        scratch_types=[pltpu.VMEM((CA,), jnp.int32),
                       pltpu.VMEM((CA, DD), jnp.float32),
                       pltpu.VMEM_SHARED((NP, DD), jnp.float32),
                       pltpu.SemaphoreType.DMA],
        interpret=_I,
    )
    return f(idx, val, zeros)[0]


def _sc_edge_body(src_h, dst_h, msg_h, zero_h, out_h,
                  sidxv, didxv, rows0, rows1, acc,
                  semz, semg0, semg1, sema0, sema1):
    cid = lax.axis_index("c")
    sid = lax.axis_index("s")
    w = _wid()
    r0 = sid * RT
    pltpu.sync_copy(zero_h.at[pl.ds(r0, RT)], acc.at[pl.ds(r0, RT)])
    pltpu.sync_copy(src_h.at[w], sidxv)
    pltpu.sync_copy(dst_h.at[w], didxv)
    plsc.subcore_barrier()

    nsub = ECW // ESUB
    bufs = (rows0, rows1)
    gsem = (semg0, semg1)
    asem = (sema0, sema1)
    gd = [None, None]
    ad = [None, None]
    for it in range(nsub):
        b = it % 2
        if ad[b] is not None:
            ad[b].wait()
        gd[b] = pltpu.async_copy(msg_h.at[sidxv.at[it]], bufs[b], gsem[b])
        if it > 0:
            pb = 1 - b
            gd[pb].wait()
            ad[pb] = pltpu.async_copy(bufs[pb], acc.at[didxv.at[it - 1]],
                                      asem[pb], add=True)
    b = (nsub - 1) % 2
    gd[b].wait()
    ad[b] = pltpu.async_copy(bufs[b], acc.at[didxv.at[nsub - 1]], asem[b],
                             add=True)
    ad[1 - b].wait()
    ad[b].wait()
    plsc.subcore_barrier()
    pltpu.sync_copy(acc.at[pl.ds(r0, RT)], out_h.at[cid, pl.ds(r0, RT)])


def _sc_edge_agg(src_e3, dst_e3, msg, zeros):
    nsub = ECW // ESUB
    f = pl.kernel(
        _sc_edge_body,
        out_type=[jax.ShapeDtypeStruct((NC, NP, DD), jnp.float32)],
        mesh=_mesh(),
        compiler_params=pltpu.CompilerParams(use_tc_tiling_on_sc=False),
        scratch_types=[pltpu.VMEM((nsub, ESUB), jnp.int32),
                       pltpu.VMEM((nsub, ESUB), jnp.int32),
                       pltpu.VMEM((ESUB, DD), jnp.float32),
                       pltpu.VMEM((ESUB, DD), jnp.float32),
                       pltpu.VMEM_SHARED((NP, DD), jnp.float32),
                       pltpu.SemaphoreType.DMA,
                       pltpu.SemaphoreType.DMA,
                       pltpu.SemaphoreType.DMA,
                       pltpu.SemaphoreType.DMA,
                       pltpu.SemaphoreType.DMA],
        interpret=_I,
    )
    return f(src_e3, dst_e3, msg, zeros)[0]


def kernel(x, edge_index, node_time, W_in, b_in, agent_emb, W_an, b_an,
           W_q, b_q, W_k, b_k, w_a, b_a, W_am1, b_am1, W_am2, b_am2,
           g_al, b_al, W_nm1, b_nm1, W_nm2, b_nm2, g_nl, b_nl, W_mv, b_mv,
           W_cm1, b_cm1, W_cm2, b_cm2, g_cl, b_cl):
    del node_time
    i32, f32 = jnp.int32, jnp.float32

    src = edge_index[0]
    dst = edge_index[1]
    order = jnp.argsort(src)
    src_s = src[order].astype(i32)
    dst_s = dst[order].astype(i32)
    nodes = jnp.arange(NN, dtype=i32)
    lo = jnp.searchsorted(src_s, nodes, side='left').astype(i32)
    hi = jnp.searchsorted(src_s, nodes, side='right').astype(i32)
    lo_p = jnp.concatenate([lo, jnp.zeros((NP - NN,), i32)])
    hi_p = jnp.concatenate([hi, jnp.zeros((NP - NN,), i32)])
    nsub = ECW // ESUB
    src_e = jnp.concatenate([src_s, jnp.zeros((EP - EE,), i32)]
                            ).reshape(NW, nsub, ESUB)
    dst_e = jnp.concatenate([dst_s, jnp.full((EP - EE,), NP - 1, i32)]
                            ).reshape(NW, nsub, ESUB)

    xp = jnp.concatenate([x, jnp.zeros((NP - NN, FF), f32)])
    zeros = jnp.zeros((NP, DD), f32)

    noises = []
    for step in range(PP):
        gkey = jax.random.fold_in(jax.random.key(42), step)
        gu = jax.random.uniform(gkey, (NN, KK), minval=1e-6, maxval=1.0 - 1e-6)
        gn = -jnp.log(-jnp.log(gu))
        noises.append(jnp.concatenate([gn, jnp.zeros((NP - NN, KK), f32)]))

    scale = 1.0 / float(DD) ** 0.5
    A = (w_a[0] * scale).reshape(1, 1).astype(f32)
    Bc = b_a[0].reshape(1, 1).astype(f32)

    r1 = lambda v: v.reshape(1, -1)
    Wk1 = W_k[:DD]
    Wk2 = W_k[DD:]

    h, s, q, t, agent = _tc_init(xp, W_in, r1(b_in), W_an, r1(b_an), Wk1,
                                 r1(agent_emb), W_q, r1(b_q), Wk2, r1(b_k))

    cur = jnp.concatenate([nodes, jnp.full((NP - NN,), NP - 1, i32)])
    visited = [nodes]
    logps = []
    for step in range(PP):
        cnt, neigh, G = _sc_gather(cur, lo_p, hi_p, dst_s, s)
        neighT = neigh.reshape(NW, KK, CA).transpose(0, 2, 1).reshape(NP, KK)
        nxt2, logp2 = _tc_select(G, neighT, cnt[:, None], cur[:, None],
                                 noises[step], q, t, A, Bc)
        nxt = nxt2[:, 0]
        visited.append(nxt[:NN])
        logps.append(logp2[:NN, 0])
        if step == PP - 1:
            break
        hsel = _sc_rowgather(nxt, h)
        agent, upd = _tc_agent(agent, hsel, W_am1, r1(b_am1), W_am2, r1(b_am2),
                               r1(g_al), r1(b_al), W_nm1, r1(b_nm1), W_nm2,
                               r1(b_nm2))
        delta = _sc_scatter_add(nxt, upd, zeros)
        hm, msg = _tc_mid(h, delta, r1(g_nl), r1(b_nl), W_mv, r1(b_mv))
        agg = _sc_edge_agg(src_e, dst_e, msg, zeros)
        h, s, q, t = _tc_final(hm, agg, agent, W_cm1, r1(b_cm1), W_cm2,
                               r1(b_cm2), r1(g_cl), r1(b_cl), W_an, r1(b_an),
                               Wk1, W_q, r1(b_q), Wk2, r1(b_k))
        cur = nxt

    return (jnp.stack(visited, axis=1), jnp.stack(logps, axis=1))


# speedup vs baseline: 7.5793x; 1.0612x over previous
import functools

import jax
import jax.numpy as jnp
from jax import lax
from jax.experimental import pallas as pl
from jax.experimental.pallas import tpu as pltpu
from jax.experimental.pallas import tpu_sc as plsc

NN = 10000
FF = 128
DD = 64
KK = 32
PP = 4
EE = 160000

NC, NS = 2, 16
NW = NC * NS
NP = 10240
CA = NP // NW
RT = NP // NS
EP = 163840
ECW = EP // NW
ESUB = 256
BB = 640
NB = NP // BB

_I = False


def _lrelu(v, s):
    return jnp.where(v >= 0, v, s * v)


def _ln(v, g, b):
    m = jnp.mean(v, axis=-1, keepdims=True)
    var = jnp.mean((v - m) ** 2, axis=-1, keepdims=True)
    return (v - m) / jnp.sqrt(var + 1e-5) * g + b


def _full(shape):
    nd = len(shape)
    return pl.BlockSpec(shape, lambda i: (0,) * nd)


def _rows(cols):
    return pl.BlockSpec((BB, cols), lambda i: (i, 0))


def _init_body(x, W_in, b_in, W_an, b_an, Wk1, emb, W_q, b_q, Wk2, b_k,
               h_o, s_o, q_o, t_o, ag_o):
    h = jnp.dot(x[...], W_in[...], preferred_element_type=jnp.float32) + b_in[...]
    hp = _lrelu(jnp.dot(h, W_an[...], preferred_element_type=jnp.float32) + b_an[...], 0.2)
    s_o[...] = jnp.dot(hp, Wk1[...], preferred_element_type=jnp.float32)
    h_o[...] = h
    ag = jnp.broadcast_to(emb[...], (BB, DD))
    q_o[...] = jnp.dot(ag, W_q[...], preferred_element_type=jnp.float32) + b_q[...]
    t_o[...] = jnp.dot(ag, Wk2[...], preferred_element_type=jnp.float32) + b_k[...]
    ag_o[...] = ag


def _tc_init(xp, W_in, b_in, W_an, b_an, Wk1, emb, W_q, b_q, Wk2, b_k):
    f32 = jnp.float32
    outs = [jax.ShapeDtypeStruct((NP, DD), f32)] * 5
    return pl.pallas_call(
        _init_body,
        grid=(NB,),
        in_specs=[_rows(FF), _full((FF, DD)), _full((1, DD)), _full((DD, DD)),
                  _full((1, DD)), _full((DD, DD)), _full((1, DD)),
                  _full((DD, DD)), _full((1, DD)), _full((DD, DD)), _full((1, DD))],
        out_specs=[_rows(DD)] * 5,
        out_shape=outs,
        interpret=_I,
    )(xp, W_in, b_in, W_an, b_an, Wk1, emb, W_q, b_q, Wk2, b_k)


def _sel_body(G, neighT, cnt, cur, noise, q, t, A, Bc, nxt_o, logp_o):
    qv = q[...]
    c = jnp.sum(qv * t[...], axis=-1, keepdims=True)
    cols = []
    for k in range(KK):
        cols.append(jnp.sum(G[k] * qv, axis=-1, keepdims=True))
    raw = jnp.concatenate(cols, axis=1)
    lg = (raw + c) * A[0, 0] + Bc[0, 0]
    kio = lax.broadcasted_iota(jnp.int32, (BB, KK), 1)
    cntv = cnt[...]
    lg = jnp.where(kio < cntv, lg, -1e9)
    y = lg + noise[...]
    mx = jnp.max(y, axis=-1, keepdims=True)
    ch = jnp.min(jnp.where(y == mx, kio, KK), axis=-1, keepdims=True)
    m2 = jnp.max(lg, axis=-1, keepdims=True)
    lse = m2 + jnp.log(jnp.sum(jnp.exp(lg - m2), axis=-1, keepdims=True))
    sel = kio == ch
    lgch = jnp.sum(jnp.where(sel, lg, 0.0), axis=-1, keepdims=True)
    nxtv = jnp.sum(jnp.where(sel, neighT[...], 0), axis=-1, keepdims=True)
    has = cntv > 0
    nxt_o[...] = jnp.where(has, nxtv, cur[...])
    logp_o[...] = jnp.where(has, lgch - lse, 0.0)


def _tc_select(G, neighT, cnt2, cur2, noise, q, t, A, Bc):
    return pl.pallas_call(
        _sel_body,
        grid=(NB,),
        in_specs=[pl.BlockSpec((KK, BB, DD), lambda i: (0, i, 0)),
                  _rows(KK), _rows(1), _rows(1), _rows(KK), _rows(DD),
                  _rows(DD), _full((1, 1)), _full((1, 1))],
        out_specs=[_rows(1), _rows(1)],
        out_shape=[jax.ShapeDtypeStruct((NP, 1), jnp.int32),
                   jax.ShapeDtypeStruct((NP, 1), jnp.float32)],
        interpret=_I,
    )(G, neighT, cnt2, cur2, noise, q, t, A, Bc)


def _agent_body(ag, hsel, W_am1, b_am1, W_am2, b_am2, g_al, b_al,
                W_nm1, b_nm1, W_nm2, b_nm2, ag_o, upd_o):
    agv = ag[...]
    hs = hsel[...]
    a_in = jnp.concatenate([agv, hs], axis=-1)
    z = jnp.dot(_lrelu(jnp.dot(a_in, W_am1[...], preferred_element_type=jnp.float32)
                       + b_am1[...], 0.01),
                W_am2[...], preferred_element_type=jnp.float32) + b_am2[...]
    ag2 = _ln(agv + z, g_al[...], b_al[...])
    n_in = jnp.concatenate([hs, ag2], axis=-1)
    upd_o[...] = jnp.dot(_lrelu(jnp.dot(n_in, W_nm1[...], preferred_element_type=jnp.float32)
                                + b_nm1[...], 0.01),
                         W_nm2[...], preferred_element_type=jnp.float32) + b_nm2[...]
    ag_o[...] = ag2


def _tc_agent(ag, hsel, W_am1, b_am1, W_am2, b_am2, g_al, b_al,
              W_nm1, b_nm1, W_nm2, b_nm2):
    f32 = jnp.float32
    return pl.pallas_call(
        _agent_body,
        grid=(NB,),
        in_specs=[_rows(DD), _rows(DD), _full((2 * DD, 2 * DD)), _full((1, 2 * DD)),
                  _full((2 * DD, DD)), _full((1, DD)), _full((1, DD)), _full((1, DD)),
                  _full((2 * DD, 2 * DD)), _full((1, 2 * DD)), _full((2 * DD, DD)),
                  _full((1, DD))],
        out_specs=[_rows(DD)] * 2,
        out_shape=[jax.ShapeDtypeStruct((NP, DD), f32)] * 2,
        interpret=_I,
    )(ag, hsel, W_am1, b_am1, W_am2, b_am2, g_al, b_al, W_nm1, b_nm1, W_nm2, b_nm2)


def _mid_body(h, delta, g_nl, b_nl, W_mv, b_mv, hm_o, msg_o):
    hm = _ln(h[...] + delta[0] + delta[1], g_nl[...], b_nl[...])
    hm_o[...] = hm
    msg_o[...] = _lrelu(jnp.dot(hm, W_mv[...], preferred_element_type=jnp.float32)
                        + b_mv[...], 0.2)


def _tc_mid(h, delta, g_nl, b_nl, W_mv, b_mv):
    f32 = jnp.float32
    return pl.pallas_call(
        _mid_body,
        grid=(NB,),
        in_specs=[_rows(DD), pl.BlockSpec((NC, BB, DD), lambda i: (0, i, 0)),
                  _full((1, DD)), _full((1, DD)), _full((DD, DD)), _full((1, DD))],
        out_specs=[_rows(DD)] * 2,
        out_shape=[jax.ShapeDtypeStruct((NP, DD), f32)] * 2,
        interpret=_I,
    )(h, delta, g_nl, b_nl, W_mv, b_mv)


def _final_body(hm, agg, ag, W_cm1, b_cm1, W_cm2, b_cm2, g_cl, b_cl,
                W_an, b_an, Wk1, W_q, b_q, Wk2, b_k,
                h_o, s_o, q_o, t_o):
    hmv = hm[...]
    c_in = jnp.concatenate([hmv, agg[0] + agg[1]], axis=-1)
    z = jnp.dot(_lrelu(jnp.dot(c_in, W_cm1[...], preferred_element_type=jnp.float32)
                       + b_cm1[...], 0.01),
                W_cm2[...], preferred_element_type=jnp.float32) + b_cm2[...]
    h = _ln(hmv + z, g_cl[...], b_cl[...])
    h_o[...] = h
    hp = _lrelu(jnp.dot(h, W_an[...], preferred_element_type=jnp.float32) + b_an[...], 0.2)
    s_o[...] = jnp.dot(hp, Wk1[...], preferred_element_type=jnp.float32)
    agv = ag[...]
    q_o[...] = jnp.dot(agv, W_q[...], preferred_element_type=jnp.float32) + b_q[...]
    t_o[...] = jnp.dot(agv, Wk2[...], preferred_element_type=jnp.float32) + b_k[...]


def _tc_final(hm, agg, ag, W_cm1, b_cm1, W_cm2, b_cm2, g_cl, b_cl,
              W_an, b_an, Wk1, W_q, b_q, Wk2, b_k):
    f32 = jnp.float32
    return pl.pallas_call(
        _final_body,
        grid=(NB,),
        in_specs=[_rows(DD), pl.BlockSpec((NC, BB, DD), lambda i: (0, i, 0)),
                  _rows(DD), _full((2 * DD, 2 * DD)), _full((1, 2 * DD)),
                  _full((2 * DD, DD)), _full((1, DD)), _full((1, DD)), _full((1, DD)),
                  _full((DD, DD)), _full((1, DD)), _full((DD, DD)),
                  _full((DD, DD)), _full((1, DD)), _full((DD, DD)), _full((1, DD))],
        out_specs=[_rows(DD)] * 4,
        out_shape=[jax.ShapeDtypeStruct((NP, DD), f32)] * 4,
        interpret=_I,
    )(hm, agg, ag, W_cm1, b_cm1, W_cm2, b_cm2, g_cl, b_cl,
      W_an, b_an, Wk1, W_q, b_q, Wk2, b_k)


@functools.cache
def _mesh():
    return plsc.VectorSubcoreMesh(core_axis_name="c", subcore_axis_name="s",
                                  num_cores=NC, num_subcores=NS)


def _wid():
    return lax.axis_index("s") * NC + lax.axis_index("c")


def _sc_gather_body(cur_h, lo_h, hi_h, dst_h, s_h,
                    cnt_h, neigh_h, G_h,
                    curv, basev, hiv, cntv, idxall, neighall,
                    rows0, rows1, rows2, rows3,
                    semi, semg0, semg1, semg2, semg3,
                    semw0, semw1, semw2, semw3):
    w = _wid()
    a0 = w * CA
    pltpu.sync_copy(cur_h.at[pl.ds(a0, CA)], curv)
    pltpu.async_copy(lo_h.at[curv], basev, semi).wait()
    pltpu.async_copy(hi_h.at[curv], hiv, semi).wait()

    def cnt_chunk(j, _):
        sl = pl.ds(j * 16, 16)
        cntv[sl] = hiv[sl] - basev[sl]
        return 0
    lax.fori_loop(0, CA // 16, cnt_chunk, 0)
    pltpu.sync_copy(cntv, cnt_h.at[pl.ds(a0, CA)])

    def idx_k(k, _):
        def idx_chunk(j, _):
            sl = pl.ds(j * 16, 16)
            idxall[pl.ds(k * CA + j * 16, 16)] = jnp.clip(basev[sl] + k, 0, EE - 1)
            return 0
        lax.fori_loop(0, CA // 16, idx_chunk, 0)
        return 0
    lax.fori_loop(0, KK, idx_k, 0)
    pltpu.async_copy(dst_h.at[idxall], neighall, semi).wait()
    pltpu.sync_copy(neighall, neigh_h.at[w])

    nb = 4
    bufs = (rows0, rows1, rows2, rows3)
    gsem = (semg0, semg1, semg2, semg3)
    wsem = (semw0, semw1, semw2, semw3)
    gd = [None] * nb
    wd = [None] * nb
    for k in range(KK):
        b = k % nb
        if wd[b] is not None:
            wd[b].wait()
        gd[b] = pltpu.async_copy(s_h.at[neighall.at[pl.ds(k * CA, CA)]],
                                 bufs[b], gsem[b])
        if k >= nb - 1:
            kp = k - (nb - 1)
            pb = kp % nb
            gd[pb].wait()
            wd[pb] = pltpu.async_copy(bufs[pb], G_h.at[kp, pl.ds(a0, CA)],
                                      wsem[pb])
    for kp in range(KK - nb + 1, KK):
        pb = kp % nb
        gd[pb].wait()
        wd[pb] = pltpu.async_copy(bufs[pb], G_h.at[kp, pl.ds(a0, CA)],
                                  wsem[pb])
    for pb in range(nb):
        wd[pb].wait()


def _sc_gather(cur, lo_p, hi_p, dst_s, s):
    i32, f32 = jnp.int32, jnp.float32
    f = pl.kernel(
        _sc_gather_body,
        out_type=[jax.ShapeDtypeStruct((NP,), i32),
                  jax.ShapeDtypeStruct((NW, KK * CA), i32),
                  jax.ShapeDtypeStruct((KK, NP, DD), f32)],
        mesh=_mesh(),
        compiler_params=pltpu.CompilerParams(use_tc_tiling_on_sc=False),
        scratch_types=[pltpu.VMEM((CA,), i32)] * 4
        + [pltpu.VMEM((KK * CA,), i32)] * 2
        + [pltpu.VMEM((CA, DD), f32)] * 4
        + [pltpu.SemaphoreType.DMA] * 9,
        interpret=_I,
    )
    return f(cur, lo_p, hi_p, dst_s, s)


def _sc_rowgather_body(idx_h, tab_h, out_h, idxv, rows, sem):
    a0 = _wid() * CA
    pltpu.sync_copy(idx_h.at[pl.ds(a0, CA)], idxv)
    pltpu.async_copy(tab_h.at[idxv], rows, sem).wait()
    pltpu.sync_copy(rows, out_h.at[pl.ds(a0, CA)])


def _sc_rowgather(idx, tab):
    f = pl.kernel(
        _sc_rowgather_body,
        out_type=[jax.ShapeDtypeStruct((NP, DD), jnp.float32)],
        mesh=_mesh(),
        compiler_params=pltpu.CompilerParams(use_tc_tiling_on_sc=False),
        scratch_types=[pltpu.VMEM((CA,), jnp.int32),
                       pltpu.VMEM((CA, DD), jnp.float32),
                       pltpu.SemaphoreType.DMA],
        interpret=_I,
    )
    return f(idx, tab)[0]


def _sc_scatter_body(idx_h, val_h, zero_h, out_h, idxv, rows, acc, sem):
    cid = lax.axis_index("c")
    sid = lax.axis_index("s")
    a0 = _wid() * CA
    r0 = sid * RT
    pltpu.sync_copy(zero_h.at[pl.ds(r0, RT)], acc.at[pl.ds(r0, RT)])
    plsc.subcore_barrier()
    pltpu.sync_copy(idx_h.at[pl.ds(a0, CA)], idxv)
    pltpu.sync_copy(val_h.at[pl.ds(a0, CA)], rows)
    pltpu.async_copy(rows, acc.at[idxv], sem, add=True).wait()
    plsc.subcore_barrier()
    pltpu.sync_copy(acc.at[pl.ds(r0, RT)], out_h.at[cid, pl.ds(r0, RT)])


def _sc_scatter_add(idx, val, zeros):
    f = pl.kernel(
        _sc_scatter_body,
        out_type=[jax.ShapeDtypeStruct((NC, NP, DD), jnp.float32)],
        mesh=_mesh(),
        compiler_params=pltpu.CompilerParams(use_tc_tiling_on_sc=False),
        scratch_types=[pltpu.VMEM((CA,), jnp.int32),
                       pltpu.VMEM((CA, DD), jnp.float32),
                       pltpu.VMEM_SHARED((NP, DD), jnp.float32),
                       pltpu.SemaphoreType.DMA],
        interpret=_I,
    )
    return f(idx, val, zeros)[0]


def _sc_edge_body(src_h, dst_h, msg_h, zero_h, out_h,
                  sidxv, didxv, rows0, rows1, rows2, rows3, acc,
                  semz, semg0, semg1, semg2, semg3,
                  sema0, sema1, sema2, sema3):
    cid = lax.axis_index("c")
    sid = lax.axis_index("s")
    w = _wid()
    r0 = sid * RT
    pltpu.sync_copy(zero_h.at[pl.ds(r0, RT)], acc.at[pl.ds(r0, RT)])
    pltpu.sync_copy(src_h.at[w], sidxv)
    pltpu.sync_copy(dst_h.at[w], didxv)
    plsc.subcore_barrier()

    nsub = ECW // ESUB
    nb = 4
    bufs = (rows0, rows1, rows2, rows3)
    gsem = (semg0, semg1, semg2, semg3)
    asem = (sema0, sema1, sema2, sema3)
    gd = [None] * nb
    ad = [None] * nb
    for it in range(nsub):
        b = it % nb
        if ad[b] is not None:
            ad[b].wait()
        gd[b] = pltpu.async_copy(msg_h.at[sidxv.at[it]], bufs[b], gsem[b])
        if it >= nb - 1:
            ip = it - (nb - 1)
            pb = ip % nb
            gd[pb].wait()
            ad[pb] = pltpu.async_copy(bufs[pb], acc.at[didxv.at[ip]],
                                      asem[pb], add=True)
    for ip in range(nsub - nb + 1, nsub):
        pb = ip % nb
        gd[pb].wait()
        ad[pb] = pltpu.async_copy(bufs[pb], acc.at[didxv.at[ip]], asem[pb],
                                  add=True)
    for pb in range(nb):
        ad[pb].wait()
    plsc.subcore_barrier()
    pltpu.sync_copy(acc.at[pl.ds(r0, RT)], out_h.at[cid, pl.ds(r0, RT)])


def _sc_edge_agg(src_e3, dst_e3, msg, zeros):
    nsub = ECW // ESUB
    f = pl.kernel(
        _sc_edge_body,
        out_type=[jax.ShapeDtypeStruct((NC, NP, DD), jnp.float32)],
        mesh=_mesh(),
        compiler_params=pltpu.CompilerParams(use_tc_tiling_on_sc=False),
        scratch_types=[pltpu.VMEM((nsub, ESUB), jnp.int32),
                       pltpu.VMEM((nsub, ESUB), jnp.int32)]
        + [pltpu.VMEM((ESUB, DD), jnp.float32)] * 4
        + [pltpu.VMEM_SHARED((NP, DD), jnp.float32)]
        + [pltpu.SemaphoreType.DMA] * 9,
        interpret=_I,
    )
    return f(src_e3, dst_e3, msg, zeros)[0]


def kernel(x, edge_index, node_time, W_in, b_in, agent_emb, W_an, b_an,
           W_q, b_q, W_k, b_k, w_a, b_a, W_am1, b_am1, W_am2, b_am2,
           g_al, b_al, W_nm1, b_nm1, W_nm2, b_nm2, g_nl, b_nl, W_mv, b_mv,
           W_cm1, b_cm1, W_cm2, b_cm2, g_cl, b_cl):
    del node_time
    i32, f32 = jnp.int32, jnp.float32

    src = edge_index[0]
    dst = edge_index[1]
    order = jnp.argsort(src)
    src_s = src[order].astype(i32)
    dst_s = dst[order].astype(i32)
    nodes = jnp.arange(NN, dtype=i32)
    lo = jnp.searchsorted(src_s, nodes, side='left').astype(i32)
    hi = jnp.searchsorted(src_s, nodes, side='right').astype(i32)
    lo_p = jnp.concatenate([lo, jnp.zeros((NP - NN,), i32)])
    hi_p = jnp.concatenate([hi, jnp.zeros((NP - NN,), i32)])
    nsub = ECW // ESUB
    src_e = jnp.concatenate([src_s, jnp.zeros((EP - EE,), i32)]
                            ).reshape(NW, nsub, ESUB)
    dst_e = jnp.concatenate([dst_s, jnp.full((EP - EE,), NP - 1, i32)]
                            ).reshape(NW, nsub, ESUB)

    xp = jnp.concatenate([x, jnp.zeros((NP - NN, FF), f32)])
    zeros = jnp.zeros((NP, DD), f32)

    noises = []
    for step in range(PP):
        gkey = jax.random.fold_in(jax.random.key(42), step)
        gu = jax.random.uniform(gkey, (NN, KK), minval=1e-6, maxval=1.0 - 1e-6)
        gn = -jnp.log(-jnp.log(gu))
        noises.append(jnp.concatenate([gn, jnp.zeros((NP - NN, KK), f32)]))

    scale = 1.0 / float(DD) ** 0.5
    A = (w_a[0] * scale).reshape(1, 1).astype(f32)
    Bc = b_a[0].reshape(1, 1).astype(f32)

    r1 = lambda v: v.reshape(1, -1)
    Wk1 = W_k[:DD]
    Wk2 = W_k[DD:]

    h, s, q, t, agent = _tc_init(xp, W_in, r1(b_in), W_an, r1(b_an), Wk1,
                                 r1(agent_emb), W_q, r1(b_q), Wk2, r1(b_k))

    cur = jnp.concatenate([nodes, jnp.full((NP - NN,), NP - 1, i32)])
    visited = [nodes]
    logps = []
    for step in range(PP):
        cnt, neigh, G = _sc_gather(cur, lo_p, hi_p, dst_s, s)
        neighT = neigh.reshape(NW, KK, CA).transpose(0, 2, 1).reshape(NP, KK)
        nxt2, logp2 = _tc_select(G, neighT, cnt[:, None], cur[:, None],
                                 noises[step], q, t, A, Bc)
        nxt = nxt2[:, 0]
        visited.append(nxt[:NN])
        logps.append(logp2[:NN, 0])
        if step == PP - 1:
            break
        hsel = _sc_rowgather(nxt, h)
        agent, upd = _tc_agent(agent, hsel, W_am1, r1(b_am1), W_am2, r1(b_am2),
                               r1(g_al), r1(b_al), W_nm1, r1(b_nm1), W_nm2,
                               r1(b_nm2))
        delta = _sc_scatter_add(nxt, upd, zeros)
        hm, msg = _tc_mid(h, delta, r1(g_nl), r1(b_nl), W_mv, r1(b_mv))
        agg = _sc_edge_agg(src_e, dst_e, msg, zeros)
        h, s, q, t = _tc_final(hm, agg, agent, W_cm1, r1(b_cm1), W_cm2,
                               r1(b_cm2), r1(g_cl), r1(b_cl), W_an, r1(b_an),
                               Wk1, W_q, r1(b_q), Wk2, r1(b_k))
        cur = nxt

    return (jnp.stack(visited, axis=1), jnp.stack(logps, axis=1))


# speedup vs baseline: 7.5870x; 1.0010x over previous
import functools

import jax
import jax.numpy as jnp
from jax import lax
from jax.experimental import pallas as pl
from jax.experimental.pallas import tpu as pltpu
from jax.experimental.pallas import tpu_sc as plsc

NN = 10000
FF = 128
DD = 64
KK = 32
PP = 4
EE = 160000

NC, NS = 2, 16
NW = NC * NS
NP = 10240
CA = NP // NW
RT = NP // NS
EP = 163840
ECW = EP // NW
ESUB = 256
BB = 640
NB = NP // BB


def _lrelu(v, s):
    return jnp.where(v >= 0, v, s * v)


def _ln(v, g, b):
    m = jnp.mean(v, axis=-1, keepdims=True)
    var = jnp.mean((v - m) ** 2, axis=-1, keepdims=True)
    return (v - m) / jnp.sqrt(var + 1e-5) * g + b


def _full(shape):
    nd = len(shape)
    return pl.BlockSpec(shape, lambda i: (0,) * nd)


def _rows(cols):
    return pl.BlockSpec((BB, cols), lambda i: (i, 0))


def _init_body(x, W_in, b_in, W_an, b_an, Wk1, emb, W_q, b_q, Wk2, b_k,
               h_o, s_o, q_o, t_o, ag_o):
    h = jnp.dot(x[...], W_in[...], preferred_element_type=jnp.float32) + b_in[...]
    hp = _lrelu(jnp.dot(h, W_an[...], preferred_element_type=jnp.float32) + b_an[...], 0.2)
    s_o[...] = jnp.dot(hp, Wk1[...], preferred_element_type=jnp.float32)
    h_o[...] = h
    ag = jnp.broadcast_to(emb[...], (BB, DD))
    q_o[...] = jnp.dot(ag, W_q[...], preferred_element_type=jnp.float32) + b_q[...]
    t_o[...] = jnp.dot(ag, Wk2[...], preferred_element_type=jnp.float32) + b_k[...]
    ag_o[...] = ag


def _tc_init(xp, W_in, b_in, W_an, b_an, Wk1, emb, W_q, b_q, Wk2, b_k):
    f32 = jnp.float32
    outs = [jax.ShapeDtypeStruct((NP, DD), f32)] * 5
    return pl.pallas_call(
        _init_body,
        grid=(NB,),
        in_specs=[_rows(FF), _full((FF, DD)), _full((1, DD)), _full((DD, DD)),
                  _full((1, DD)), _full((DD, DD)), _full((1, DD)),
                  _full((DD, DD)), _full((1, DD)), _full((DD, DD)), _full((1, DD))],
        out_specs=[_rows(DD)] * 5,
        out_shape=outs,
    )(xp, W_in, b_in, W_an, b_an, Wk1, emb, W_q, b_q, Wk2, b_k)


def _sel_body(G, neighT, cnt, cur, noise, q, t, A, Bc, nxt_o, logp_o):
    qv = q[...]
    c = jnp.sum(qv * t[...], axis=-1, keepdims=True)
    cols = []
    for k in range(KK):
        cols.append(jnp.sum(G[k] * qv, axis=-1, keepdims=True))
    raw = jnp.concatenate(cols, axis=1)
    lg = (raw + c) * A[0, 0] + Bc[0, 0]
    kio = lax.broadcasted_iota(jnp.int32, (BB, KK), 1)
    cntv = cnt[...]
    lg = jnp.where(kio < cntv, lg, -1e9)
    y = lg + noise[...]
    mx = jnp.max(y, axis=-1, keepdims=True)
    ch = jnp.min(jnp.where(y == mx, kio, KK), axis=-1, keepdims=True)
    m2 = jnp.max(lg, axis=-1, keepdims=True)
    lse = m2 + jnp.log(jnp.sum(jnp.exp(lg - m2), axis=-1, keepdims=True))
    sel = kio == ch
    lgch = jnp.sum(jnp.where(sel, lg, 0.0), axis=-1, keepdims=True)
    nxtv = jnp.sum(jnp.where(sel, neighT[...], 0), axis=-1, keepdims=True)
    has = cntv > 0
    nxt_o[...] = jnp.where(has, nxtv, cur[...])
    logp_o[...] = jnp.where(has, lgch - lse, 0.0)


def _tc_select(G, neighT, cnt2, cur2, noise, q, t, A, Bc):
    return pl.pallas_call(
        _sel_body,
        grid=(NB,),
        in_specs=[pl.BlockSpec((KK, BB, DD), lambda i: (0, i, 0)),
                  _rows(KK), _rows(1), _rows(1), _rows(KK), _rows(DD),
                  _rows(DD), _full((1, 1)), _full((1, 1))],
        out_specs=[_rows(1), _rows(1)],
        out_shape=[jax.ShapeDtypeStruct((NP, 1), jnp.int32),
                   jax.ShapeDtypeStruct((NP, 1), jnp.float32)],
    )(G, neighT, cnt2, cur2, noise, q, t, A, Bc)


def _agent_body(ag, hsel, W_am1, b_am1, W_am2, b_am2, g_al, b_al,
                W_nm1, b_nm1, W_nm2, b_nm2, ag_o, upd_o):
    agv = ag[...]
    hs = hsel[...]
    a_in = jnp.concatenate([agv, hs], axis=-1)
    z = jnp.dot(_lrelu(jnp.dot(a_in, W_am1[...], preferred_element_type=jnp.float32)
                       + b_am1[...], 0.01),
                W_am2[...], preferred_element_type=jnp.float32) + b_am2[...]
    ag2 = _ln(agv + z, g_al[...], b_al[...])
    n_in = jnp.concatenate([hs, ag2], axis=-1)
    upd_o[...] = jnp.dot(_lrelu(jnp.dot(n_in, W_nm1[...], preferred_element_type=jnp.float32)
                                + b_nm1[...], 0.01),
                         W_nm2[...], preferred_element_type=jnp.float32) + b_nm2[...]
    ag_o[...] = ag2


def _tc_agent(ag, hsel, W_am1, b_am1, W_am2, b_am2, g_al, b_al,
              W_nm1, b_nm1, W_nm2, b_nm2):
    f32 = jnp.float32
    return pl.pallas_call(
        _agent_body,
        grid=(NB,),
        in_specs=[_rows(DD), _rows(DD), _full((2 * DD, 2 * DD)), _full((1, 2 * DD)),
                  _full((2 * DD, DD)), _full((1, DD)), _full((1, DD)), _full((1, DD)),
                  _full((2 * DD, 2 * DD)), _full((1, 2 * DD)), _full((2 * DD, DD)),
                  _full((1, DD))],
        out_specs=[_rows(DD)] * 2,
        out_shape=[jax.ShapeDtypeStruct((NP, DD), f32)] * 2,
    )(ag, hsel, W_am1, b_am1, W_am2, b_am2, g_al, b_al, W_nm1, b_nm1, W_nm2, b_nm2)


def _mid_body(h, delta, g_nl, b_nl, W_mv, b_mv, hm_o, msg_o):
    hm = _ln(h[...] + delta[0] + delta[1], g_nl[...], b_nl[...])
    hm_o[...] = hm
    msg_o[...] = _lrelu(jnp.dot(hm, W_mv[...], preferred_element_type=jnp.float32)
                        + b_mv[...], 0.2)


def _tc_mid(h, delta, g_nl, b_nl, W_mv, b_mv):
    f32 = jnp.float32
    return pl.pallas_call(
        _mid_body,
        grid=(NB,),
        in_specs=[_rows(DD), pl.BlockSpec((NC, BB, DD), lambda i: (0, i, 0)),
                  _full((1, DD)), _full((1, DD)), _full((DD, DD)), _full((1, DD))],
        out_specs=[_rows(DD)] * 2,
        out_shape=[jax.ShapeDtypeStruct((NP, DD), f32)] * 2,
    )(h, delta, g_nl, b_nl, W_mv, b_mv)


def _final_body(hm, agg, ag, W_cm1, b_cm1, W_cm2, b_cm2, g_cl, b_cl,
                W_an, b_an, Wk1, W_q, b_q, Wk2, b_k,
                h_o, s_o, q_o, t_o):
    hmv = hm[...]
    c_in = jnp.concatenate([hmv, agg[0] + agg[1]], axis=-1)
    z = jnp.dot(_lrelu(jnp.dot(c_in, W_cm1[...], preferred_element_type=jnp.float32)
                       + b_cm1[...], 0.01),
                W_cm2[...], preferred_element_type=jnp.float32) + b_cm2[...]
    h = _ln(hmv + z, g_cl[...], b_cl[...])
    h_o[...] = h
    hp = _lrelu(jnp.dot(h, W_an[...], preferred_element_type=jnp.float32) + b_an[...], 0.2)
    s_o[...] = jnp.dot(hp, Wk1[...], preferred_element_type=jnp.float32)
    agv = ag[...]
    q_o[...] = jnp.dot(agv, W_q[...], preferred_element_type=jnp.float32) + b_q[...]
    t_o[...] = jnp.dot(agv, Wk2[...], preferred_element_type=jnp.float32) + b_k[...]


def _tc_final(hm, agg, ag, W_cm1, b_cm1, W_cm2, b_cm2, g_cl, b_cl,
              W_an, b_an, Wk1, W_q, b_q, Wk2, b_k):
    f32 = jnp.float32
    return pl.pallas_call(
        _final_body,
        grid=(NB,),
        in_specs=[_rows(DD), pl.BlockSpec((NC, BB, DD), lambda i: (0, i, 0)),
                  _rows(DD), _full((2 * DD, 2 * DD)), _full((1, 2 * DD)),
                  _full((2 * DD, DD)), _full((1, DD)), _full((1, DD)), _full((1, DD)),
                  _full((DD, DD)), _full((1, DD)), _full((DD, DD)),
                  _full((DD, DD)), _full((1, DD)), _full((DD, DD)), _full((1, DD))],
        out_specs=[_rows(DD)] * 4,
        out_shape=[jax.ShapeDtypeStruct((NP, DD), f32)] * 4,
    )(hm, agg, ag, W_cm1, b_cm1, W_cm2, b_cm2, g_cl, b_cl,
      W_an, b_an, Wk1, W_q, b_q, Wk2, b_k)


@functools.cache
def _mesh():
    return plsc.VectorSubcoreMesh(core_axis_name="c", subcore_axis_name="s",
                                  num_cores=NC, num_subcores=NS)


def _wid():
    return lax.axis_index("s") * NC + lax.axis_index("c")


def _sc_gather_body(cur_h, lo_h, hi_h, dst_h, s_h,
                    cnt_h, neigh_h, G_h,
                    curv, basev, hiv, cntv, idxall, neighall,
                    rows0, rows1, rows2, rows3,
                    semi, semg0, semg1, semg2, semg3,
                    semw0, semw1, semw2, semw3):
    w = _wid()
    a0 = w * CA
    pltpu.sync_copy(cur_h.at[pl.ds(a0, CA)], curv)
    pltpu.async_copy(lo_h.at[curv], basev, semi).wait()
    pltpu.async_copy(hi_h.at[curv], hiv, semi).wait()

    def cnt_chunk(j, _):
        sl = pl.ds(j * 16, 16)
        cntv[sl] = hiv[sl] - basev[sl]
        return 0
    lax.fori_loop(0, CA // 16, cnt_chunk, 0)
    pltpu.sync_copy(cntv, cnt_h.at[pl.ds(a0, CA)])

    def idx_k(k, _):
        def idx_chunk(j, _):
            sl = pl.ds(j * 16, 16)
            idxall[pl.ds(k * CA + j * 16, 16)] = jnp.clip(basev[sl] + k, 0, EE - 1)
            return 0
        lax.fori_loop(0, CA // 16, idx_chunk, 0)
        return 0
    lax.fori_loop(0, KK, idx_k, 0)
    pltpu.async_copy(dst_h.at[idxall], neighall, semi).wait()
    pltpu.sync_copy(neighall, neigh_h.at[w])

    nb = 4
    bufs = (rows0, rows1, rows2, rows3)
    gsem = (semg0, semg1, semg2, semg3)
    wsem = (semw0, semw1, semw2, semw3)
    gd = [None] * nb
    wd = [None] * nb
    for k in range(KK):
        b = k % nb
        if wd[b] is not None:
            wd[b].wait()
        gd[b] = pltpu.async_copy(s_h.at[neighall.at[pl.ds(k * CA, CA)]],
                                 bufs[b], gsem[b])
        if k >= nb - 1:
            kp = k - (nb - 1)
            pb = kp % nb
            gd[pb].wait()
            wd[pb] = pltpu.async_copy(bufs[pb], G_h.at[kp, pl.ds(a0, CA)],
                                      wsem[pb])
    for kp in range(KK - nb + 1, KK):
        pb = kp % nb
        gd[pb].wait()
        wd[pb] = pltpu.async_copy(bufs[pb], G_h.at[kp, pl.ds(a0, CA)],
                                  wsem[pb])
    for pb in range(nb):
        wd[pb].wait()


def _sc_gather(cur, lo_p, hi_p, dst_s, s):
    i32, f32 = jnp.int32, jnp.float32
    f = pl.kernel(
        _sc_gather_body,
        out_type=[jax.ShapeDtypeStruct((NP,), i32),
                  jax.ShapeDtypeStruct((NW, KK * CA), i32),
                  jax.ShapeDtypeStruct((KK, NP, DD), f32)],
        mesh=_mesh(),
        compiler_params=pltpu.CompilerParams(use_tc_tiling_on_sc=False),
        scratch_types=[pltpu.VMEM((CA,), i32)] * 4
        + [pltpu.VMEM((KK * CA,), i32)] * 2
        + [pltpu.VMEM((CA, DD), f32)] * 4
        + [pltpu.SemaphoreType.DMA] * 9,
    )
    return f(cur, lo_p, hi_p, dst_s, s)


def _sc_rowgather_body(idx_h, tab_h, out_h, idxv, rows, sem):
    a0 = _wid() * CA
    pltpu.sync_copy(idx_h.at[pl.ds(a0, CA)], idxv)
    pltpu.async_copy(tab_h.at[idxv], rows, sem).wait()
    pltpu.sync_copy(rows, out_h.at[pl.ds(a0, CA)])


def _sc_rowgather(idx, tab):
    f = pl.kernel(
        _sc_rowgather_body,
        out_type=[jax.ShapeDtypeStruct((NP, DD), jnp.float32)],
        mesh=_mesh(),
        compiler_params=pltpu.CompilerParams(use_tc_tiling_on_sc=False),
        scratch_types=[pltpu.VMEM((CA,), jnp.int32),
                       pltpu.VMEM((CA, DD), jnp.float32),
                       pltpu.SemaphoreType.DMA],
    )
    return f(idx, tab)[0]


def _sc_scatter_body(idx_h, val_h, zero_h, out_h, idxv, rows, acc, sem):
    cid = lax.axis_index("c")
    sid = lax.axis_index("s")
    a0 = _wid() * CA
    r0 = sid * RT
    pltpu.sync_copy(zero_h.at[pl.ds(r0, RT)], acc.at[pl.ds(r0, RT)])
    plsc.subcore_barrier()
    pltpu.sync_copy(idx_h.at[pl.ds(a0, CA)], idxv)
    pltpu.sync_copy(val_h.at[pl.ds(a0, CA)], rows)
    pltpu.async_copy(rows, acc.at[idxv], sem, add=True).wait()
    plsc.subcore_barrier()
    pltpu.sync_copy(acc.at[pl.ds(r0, RT)], out_h.at[cid, pl.ds(r0, RT)])


def _sc_scatter_add(idx, val, zeros):
    f = pl.kernel(
        _sc_scatter_body,
        out_type=[jax.ShapeDtypeStruct((NC, NP, DD), jnp.float32)],
        mesh=_mesh(),
        compiler_params=pltpu.CompilerParams(use_tc_tiling_on_sc=False),
        scratch_types=[pltpu.VMEM((CA,), jnp.int32),
                       pltpu.VMEM((CA, DD), jnp.float32),
                       pltpu.VMEM_SHARED((NP, DD), jnp.float32),
                       pltpu.SemaphoreType.DMA],
    )
    return f(idx, val, zeros)[0]


def _sc_edge_body(src_h, dst_h, msg_h, zero_h, out_h,
                  sidxv, didxv, rows0, rows1, rows2, rows3, acc,
                  semz, semg0, semg1, semg2, semg3,
                  sema0, sema1, sema2, sema3):
    cid = lax.axis_index("c")
    sid = lax.axis_index("s")
    w = _wid()
    r0 = sid * RT
    pltpu.sync_copy(zero_h.at[pl.ds(r0, RT)], acc.at[pl.ds(r0, RT)])
    pltpu.sync_copy(src_h.at[w], sidxv)
    pltpu.sync_copy(dst_h.at[w], didxv)
    plsc.subcore_barrier()

    nsub = ECW // ESUB
    nb = 4
    bufs = (rows0, rows1, rows2, rows3)
    gsem = (semg0, semg1, semg2, semg3)
    asem = (sema0, sema1, sema2, sema3)
    gd = [None] * nb
    ad = [None] * nb
    for it in range(nsub):
        b = it % nb
        if ad[b] is not None:
            ad[b].wait()
        gd[b] = pltpu.async_copy(msg_h.at[sidxv.at[it]], bufs[b], gsem[b])
        if it >= nb - 1:
            ip = it - (nb - 1)
            pb = ip % nb
            gd[pb].wait()
            ad[pb] = pltpu.async_copy(bufs[pb], acc.at[didxv.at[ip]],
                                      asem[pb], add=True)
    for ip in range(nsub - nb + 1, nsub):
        pb = ip % nb
        gd[pb].wait()
        ad[pb] = pltpu.async_copy(bufs[pb], acc.at[didxv.at[ip]], asem[pb],
                                  add=True)
    for pb in range(nb):
        ad[pb].wait()
    plsc.subcore_barrier()
    pltpu.sync_copy(acc.at[pl.ds(r0, RT)], out_h.at[cid, pl.ds(r0, RT)])


def _sc_edge_agg(src_e3, dst_e3, msg, zeros):
    nsub = ECW // ESUB
    f = pl.kernel(
        _sc_edge_body,
        out_type=[jax.ShapeDtypeStruct((NC, NP, DD), jnp.float32)],
        mesh=_mesh(),
        compiler_params=pltpu.CompilerParams(use_tc_tiling_on_sc=False),
        scratch_types=[pltpu.VMEM((nsub, ESUB), jnp.int32),
                       pltpu.VMEM((nsub, ESUB), jnp.int32)]
        + [pltpu.VMEM((ESUB, DD), jnp.float32)] * 4
        + [pltpu.VMEM_SHARED((NP, DD), jnp.float32)]
        + [pltpu.SemaphoreType.DMA] * 9,
    )
    return f(src_e3, dst_e3, msg, zeros)[0]


def kernel(x, edge_index, node_time, W_in, b_in, agent_emb, W_an, b_an,
           W_q, b_q, W_k, b_k, w_a, b_a, W_am1, b_am1, W_am2, b_am2,
           g_al, b_al, W_nm1, b_nm1, W_nm2, b_nm2, g_nl, b_nl, W_mv, b_mv,
           W_cm1, b_cm1, W_cm2, b_cm2, g_cl, b_cl):
    del node_time
    i32, f32 = jnp.int32, jnp.float32

    src = edge_index[0]
    dst = edge_index[1]
    order = jnp.argsort(src)
    src_s = src[order].astype(i32)
    dst_s = dst[order].astype(i32)
    nodes = jnp.arange(NN, dtype=i32)
    lo = jnp.searchsorted(src_s, nodes, side='left').astype(i32)
    hi = jnp.searchsorted(src_s, nodes, side='right').astype(i32)
    lo_p = jnp.concatenate([lo, jnp.zeros((NP - NN,), i32)])
    hi_p = jnp.concatenate([hi, jnp.zeros((NP - NN,), i32)])
    nsub = ECW // ESUB
    src_e = jnp.concatenate([src_s, jnp.zeros((EP - EE,), i32)]
                            ).reshape(NW, nsub, ESUB)
    dst_e = jnp.concatenate([dst_s, jnp.full((EP - EE,), NP - 1, i32)]
                            ).reshape(NW, nsub, ESUB)

    xp = jnp.concatenate([x, jnp.zeros((NP - NN, FF), f32)])
    zeros = jnp.zeros((NP, DD), f32)

    noises = []
    for step in range(PP):
        gkey = jax.random.fold_in(jax.random.key(42), step)
        gu = jax.random.uniform(gkey, (NN, KK), minval=1e-6, maxval=1.0 - 1e-6)
        gn = -jnp.log(-jnp.log(gu))
        noises.append(jnp.concatenate([gn, jnp.zeros((NP - NN, KK), f32)]))

    scale = 1.0 / float(DD) ** 0.5
    A = (w_a[0] * scale).reshape(1, 1).astype(f32)
    Bc = b_a[0].reshape(1, 1).astype(f32)

    r1 = lambda v: v.reshape(1, -1)
    Wk1 = W_k[:DD]
    Wk2 = W_k[DD:]

    h, s, q, t, agent = _tc_init(xp, W_in, r1(b_in), W_an, r1(b_an), Wk1,
                                 r1(agent_emb), W_q, r1(b_q), Wk2, r1(b_k))

    cur = jnp.concatenate([nodes, jnp.full((NP - NN,), NP - 1, i32)])
    visited = [nodes]
    logps = []
    for step in range(PP):
        cnt, neigh, G = _sc_gather(cur, lo_p, hi_p, dst_s, s)
        neighT = neigh.reshape(NW, KK, CA).transpose(0, 2, 1).reshape(NP, KK)
        nxt2, logp2 = _tc_select(G, neighT, cnt[:, None], cur[:, None],
                                 noises[step], q, t, A, Bc)
        nxt = nxt2[:, 0]
        visited.append(nxt[:NN])
        logps.append(logp2[:NN, 0])
        if step == PP - 1:
            break
        hsel = _sc_rowgather(nxt, h)
        agent, upd = _tc_agent(agent, hsel, W_am1, r1(b_am1), W_am2, r1(b_am2),
                               r1(g_al), r1(b_al), W_nm1, r1(b_nm1), W_nm2,
                               r1(b_nm2))
        delta = _sc_scatter_add(nxt, upd, zeros)
        hm, msg = _tc_mid(h, delta, r1(g_nl), r1(b_nl), W_mv, r1(b_mv))
        agg = _sc_edge_agg(src_e, dst_e, msg, zeros)
        h, s, q, t = _tc_final(hm, agg, agent, W_cm1, r1(b_cm1), W_cm2,
                               r1(b_cm2), r1(g_cl), r1(b_cl), W_an, r1(b_an),
                               Wk1, W_q, r1(b_q), Wk2, r1(b_k))
        cur = nxt

    return (jnp.stack(visited, axis=1), jnp.stack(logps, axis=1))
